# baseline (speedup 1.0000x reference)
import jax
import jax.numpy as jnp
from jax.experimental import pallas as pl
from jax.experimental.pallas import tpu as pltpu

_NBUF = 2
_CHUNK = 1024


def _silu_stream(x_hbm, o_hbm, in_bufs, out_bufs, in_sems, out_sems):
    i = pl.program_id(0)
    nsteps = pl.num_programs(0)
    slot = jax.lax.rem(i, _NBUF)

    def _in_copy(step, buf):
        return pltpu.make_async_copy(
            x_hbm.at[pl.ds(step * _CHUNK, _CHUNK)], in_bufs.at[buf], in_sems.at[buf]
        )

    def _out_copy(step, buf):
        return pltpu.make_async_copy(
            out_bufs.at[buf], o_hbm.at[pl.ds(step * _CHUNK, _CHUNK)], out_sems.at[buf]
        )

    @pl.when(i == 0)
    def _prologue():
        for b in range(_NBUF):
            _in_copy(b, b).start()

    _in_copy(i, slot).wait()

    @pl.when(i >= _NBUF)
    def _wait_prev_out():
        _out_copy(i, slot).wait()

    xv = in_bufs[slot]
    out_bufs[slot] = xv * jax.nn.sigmoid(xv)

    _out_copy(i, slot).start()

    @pl.when(i + _NBUF < nsteps)
    def _next_in():
        _in_copy(i + _NBUF, slot).start()

    @pl.when(i == nsteps - 1)
    def _epilogue():
        for b in range(_NBUF):
            _out_copy(0, b).wait()


def kernel(x, logit_decay, log_tau, log_beta, log_gamma):
    del logit_decay, log_tau, log_beta, log_gamma
    b, t, d = x.shape
    rows = b * t
    xf = x.reshape(rows, d)
    nsteps = rows // _CHUNK
    out = pl.pallas_call(
        _silu_stream,
        grid=(nsteps,),
        in_specs=[pl.BlockSpec(memory_space=pltpu.MemorySpace.HBM)],
        out_specs=pl.BlockSpec(memory_space=pltpu.MemorySpace.HBM),
        out_shape=jax.ShapeDtypeStruct((rows, d), x.dtype),
        scratch_shapes=[
            pltpu.VMEM((_NBUF, _CHUNK, d), x.dtype),
            pltpu.VMEM((_NBUF, _CHUNK, d), x.dtype),
            pltpu.SemaphoreType.DMA((_NBUF,)),
            pltpu.SemaphoreType.DMA((_NBUF,)),
        ],
    )(xf)
    return out.reshape(b, t, d)


# speedup vs baseline: 1.0654x; 1.0654x over previous
import jax
import jax.numpy as jnp
from jax.experimental import pallas as pl
from jax.experimental.pallas import tpu as pltpu

_NBUF = 3
_CHUNK = 1024


def _silu_stream(x_hbm, o_hbm, in_bufs, out_bufs, in_sems, out_sems):
    i = pl.program_id(0)
    nsteps = pl.num_programs(0)
    slot = jax.lax.rem(i, _NBUF)

    _H = _CHUNK // 2

    class _Pair:
        def __init__(self, copies):
            self.copies = copies

        def start(self):
            for c in self.copies:
                c.start()

        def wait(self):
            for c in self.copies:
                c.wait()

    def _in_copy(step, buf):
        return _Pair([
            pltpu.make_async_copy(
                x_hbm.at[pl.ds(step * _CHUNK + h * _H, _H)],
                in_bufs.at[buf, pl.ds(h * _H, _H)],
                in_sems.at[buf],
            )
            for h in range(2)
        ])

    def _out_copy(step, buf):
        return _Pair([
            pltpu.make_async_copy(
                out_bufs.at[buf, pl.ds(h * _H, _H)],
                o_hbm.at[pl.ds(step * _CHUNK + h * _H, _H)],
                out_sems.at[buf],
            )
            for h in range(2)
        ])

    @pl.when(i == 0)
    def _prologue():
        for b in range(_NBUF):
            _in_copy(b, b).start()

    _in_copy(i, slot).wait()

    @pl.when(i >= _NBUF)
    def _wait_prev_out():
        _out_copy(i, slot).wait()

    xv = in_bufs[slot]
    out_bufs[slot] = xv * jax.nn.sigmoid(xv)

    _out_copy(i, slot).start()

    @pl.when(i + _NBUF < nsteps)
    def _next_in():
        _in_copy(i + _NBUF, slot).start()

    @pl.when(i == nsteps - 1)
    def _epilogue():
        for b in range(_NBUF):
            _out_copy(0, b).wait()


def kernel(x, logit_decay, log_tau, log_beta, log_gamma):
    del logit_decay, log_tau, log_beta, log_gamma
    b, t, d = x.shape
    rows = b * t
    xf = x.reshape(rows, d)
    nsteps = rows // _CHUNK
    out = pl.pallas_call(
        _silu_stream,
        grid=(nsteps,),
        in_specs=[pl.BlockSpec(memory_space=pltpu.MemorySpace.HBM)],
        out_specs=pl.BlockSpec(memory_space=pltpu.MemorySpace.HBM),
        out_shape=jax.ShapeDtypeStruct((rows, d), x.dtype),
        scratch_shapes=[
            pltpu.VMEM((_NBUF, _CHUNK, d), x.dtype),
            pltpu.VMEM((_NBUF, _CHUNK, d), x.dtype),
            pltpu.SemaphoreType.DMA((_NBUF,)),
            pltpu.SemaphoreType.DMA((_NBUF,)),
        ],
    )(xf)
    return out.reshape(b, t, d)


# speedup vs baseline: 1.0691x; 1.0034x over previous
import jax
import jax.numpy as jnp
from jax.experimental import pallas as pl
from jax.experimental.pallas import tpu as pltpu

_NBUF = 3
_CHUNK = 1024


def _silu_stream(x_hbm, o_hbm, in_bufs, out_bufs, in_sems, out_sems):
    i = pl.program_id(0)
    nsteps = pl.num_programs(0)
    slot = jax.lax.rem(i, _NBUF)

    _H = _CHUNK // 4

    class _Pair:
        def __init__(self, copies):
            self.copies = copies

        def start(self):
            for c in self.copies:
                c.start()

        def wait(self):
            for c in self.copies:
                c.wait()

    def _in_copy(step, buf):
        return _Pair([
            pltpu.make_async_copy(
                x_hbm.at[pl.ds(step * _CHUNK + h * _H, _H)],
                in_bufs.at[buf, pl.ds(h * _H, _H)],
                in_sems.at[buf],
            )
            for h in range(4)
        ])

    def _out_copy(step, buf):
        return _Pair([
            pltpu.make_async_copy(
                out_bufs.at[buf, pl.ds(h * _H, _H)],
                o_hbm.at[pl.ds(step * _CHUNK + h * _H, _H)],
                out_sems.at[buf],
            )
            for h in range(4)
        ])

    @pl.when(i == 0)
    def _prologue():
        for b in range(_NBUF):
            _in_copy(b, b).start()

    _in_copy(i, slot).wait()

    @pl.when(i >= _NBUF)
    def _wait_prev_out():
        _out_copy(i, slot).wait()

    xv = in_bufs[slot]
    out_bufs[slot] = xv * jax.nn.sigmoid(xv)

    _out_copy(i, slot).start()

    @pl.when(i + _NBUF < nsteps)
    def _next_in():
        _in_copy(i + _NBUF, slot).start()

    @pl.when(i == nsteps - 1)
    def _epilogue():
        for b in range(_NBUF):
            _out_copy(0, b).wait()


def kernel(x, logit_decay, log_tau, log_beta, log_gamma):
    del logit_decay, log_tau, log_beta, log_gamma
    b, t, d = x.shape
    rows = b * t
    xf = x.reshape(rows, d)
    nsteps = rows // _CHUNK
    out = pl.pallas_call(
        _silu_stream,
        grid=(nsteps,),
        in_specs=[pl.BlockSpec(memory_space=pltpu.MemorySpace.HBM)],
        out_specs=pl.BlockSpec(memory_space=pltpu.MemorySpace.HBM),
        out_shape=jax.ShapeDtypeStruct((rows, d), x.dtype),
        scratch_shapes=[
            pltpu.VMEM((_NBUF, _CHUNK, d), x.dtype),
            pltpu.VMEM((_NBUF, _CHUNK, d), x.dtype),
            pltpu.SemaphoreType.DMA((_NBUF,)),
            pltpu.SemaphoreType.DMA((_NBUF,)),
        ],
    )(xf)
    return out.reshape(b, t, d)
